# baseline (speedup 1.0000x reference)
import functools

import jax
import jax.numpy as jnp
from jax import lax
from jax.experimental import pallas as pl
from jax.experimental.pallas import tpu as pltpu
from jax.experimental.pallas import tpu_sc as plsc

_SEC_BLK = 2560


def _mega_body(fs_ref, fd_ref, ewt_ref, we_ref, u_ref, v_ref, t_ref, z_ref):
    i = pl.program_id(0)

    @pl.when(i < 4)
    def _():
        z_ref[...] = jnp.dot(fd_ref[...], u_ref[...],
                             preferred_element_type=jnp.float32)

    @pl.when((i >= 4) & (i < 8))
    def _():
        t_ref[...] = jnp.dot(fs_ref[...], u_ref[...],
                             preferred_element_type=jnp.float32)

    @pl.when((i >= 8) & (i < 12))
    def _():
        t_ref[...] = jnp.dot(fd_ref[...], v_ref[...],
                             preferred_element_type=jnp.float32)

    @pl.when(i >= 12)
    def _():
        t_ref[...] = jnp.dot(ewt_ref[...], we_ref[...],
                             preferred_element_type=jnp.float32)


def _final_body(z_ref, p_ref, fd_ref, g_ref, b_ref, o_ref):
    x = z_ref[...] + p_ref[0] + p_ref[1]
    mean = jnp.mean(x, axis=-1, keepdims=True)
    xc = x - mean
    var = jnp.mean(xc * xc, axis=-1, keepdims=True)
    y = xc * lax.rsqrt(var + 1e-5)
    o_ref[...] = y * g_ref[...] + b_ref[...] + fd_ref[...]


def _make_sc_edge_call(n_nodes, n_edges, d, n_sec):
    info = plsc.get_sparse_core_info()
    nc, ns, lanes = info.num_cores, info.num_subcores, info.num_lanes
    nw = nc * ns
    assert n_edges % nw == 0
    epw = n_edges // nw
    blk = 16
    assert blk == lanes
    nchunk = epw // blk
    n_pad = n_sec
    rows_per_tile = n_pad // ns
    zrows = 40
    assert rows_per_tile % zrows == 0
    nvec = d // lanes
    nring = 3
    hd_base = n_sec
    ew_base = 2 * n_sec

    nsteady = (nchunk - (nring + 1)) // nring
    tail = list(range(nring * nsteady, nchunk))

    mesh = plsc.VectorSubcoreMesh(core_axis_name="c", subcore_axis_name="s")

    scratch = (
        [pltpu.VMEM((epw,), jnp.int32)] * 2 +
        [pltpu.VMEM((3 * blk, d), jnp.float32)] * nring +
        [pltpu.VMEM((3 * blk,), jnp.int32)] * nring +
        [pltpu.VMEM((blk,), jnp.int32)] * nring +
        [pltpu.VMEM_SHARED((n_pad, d), jnp.float32)] +
        [pltpu.SemaphoreType.DMA] * (2 * nring)
    )

    @functools.partial(
        pl.kernel,
        out_type=jax.ShapeDtypeStruct((nc, n_pad, d), jnp.float32),
        mesh=mesh,
        scratch_types=scratch,
    )
    def sc_edge(t_hbm, src_hbm, dst_hbm, out_hbm, *scr):
        src_v, dst_v = scr[0], scr[1]
        bufs = scr[2:2 + nring]
        ibufs = scr[2 + nring:2 + 2 * nring]
        dbufs = scr[2 + 2 * nring:2 + 3 * nring]
        aggr_sh = scr[2 + 3 * nring]
        sems = scr[3 + 3 * nring:3 + 4 * nring]
        ssems = scr[3 + 4 * nring:3 + 5 * nring]

        c = lax.axis_index("c")
        s = lax.axis_index("s")
        wid = c * ns + s
        base_edge = wid * epw

        @plsc.parallel_loop(0, zrows)
        def _zero_row(i):
            for j in range(nvec):
                bufs[0][i, pl.ds(j * lanes, lanes)] = jnp.zeros(
                    (lanes,), jnp.float32)
        for k in range(rows_per_tile // zrows):
            pltpu.sync_copy(
                bufs[0].at[pl.ds(0, zrows)],
                aggr_sh.at[pl.ds(s * rows_per_tile + k * zrows, zrows)])
        plsc.subcore_barrier()

        pltpu.sync_copy(src_hbm.at[pl.ds(base_edge, epw)], src_v)
        pltpu.sync_copy(dst_hbm.at[pl.ds(base_edge, epw)], dst_v)

        iota16 = lax.iota(jnp.int32, blk)

        def drain_scatter(b):
            pltpu.make_async_copy(bufs[b].at[pl.ds(blk, blk)],
                                  aggr_sh.at[dbufs[b]], ssems[b]).wait()

        def start(ch, b):
            s_idx = src_v[pl.ds(ch * blk, blk)]
            d_idx = dst_v[pl.ds(ch * blk, blk)]
            ibufs[b][pl.ds(0, blk)] = s_idx
            ibufs[b][pl.ds(blk, blk)] = d_idx + hd_base
            ibufs[b][pl.ds(2 * blk, blk)] = (
                iota16 + (ew_base + base_edge + ch * blk))
            dbufs[b][...] = d_idx
            pltpu.async_copy(t_hbm.at[ibufs[b]], bufs[b], sems[b])

        def wait(b):
            pltpu.make_async_copy(t_hbm.at[ibufs[b]], bufs[b],
                                  sems[b]).wait()

        def process(b):
            buf = bufs[b]

            @plsc.parallel_loop(0, blk, unroll=2)
            def _row(i):
                for j in range(nvec):
                    sl = pl.ds(j * lanes, lanes)
                    a = buf[i, sl]
                    x = a * buf[blk + i, sl] * buf[2 * blk + i, sl]
                    r = 1.0 + jnp.exp(-x)
                    buf[blk + i, sl] = a / r

            pltpu.async_copy(buf.at[pl.ds(blk, blk)],
                             aggr_sh.at[dbufs[b]], ssems[b], add=True)

        for q in range(nring - 1):
            start(q, q)

        for q in range(nring):
            wait(q)
            nxt = q + nring - 1
            nslot = (q + nring - 1) % nring
            if nxt >= nring:
                drain_scatter(nslot)
            start(nxt, nslot)
            process(q)

        def ring_body(t, carry):
            for q in range(nring):
                ch = nring * t + q
                wait(q)
                nslot = (q + nring - 1) % nring
                drain_scatter(nslot)
                start(ch + nring - 1, nslot)
                process(q)
            return carry
        lax.fori_loop(1, nsteady, ring_body, 0)

        for ch in tail:
            b = ch % nring
            wait(b)
            nxt = ch + nring - 1
            if nxt < nchunk:
                nslot = nxt % nring
                drain_scatter(nslot)
                start(nxt, nslot)
            process(b)

        for b in range(nring):
            drain_scatter(b)

        plsc.subcore_barrier()
        pltpu.sync_copy(
            aggr_sh.at[pl.ds(s * rows_per_tile, rows_per_tile)],
            out_hbm.at[c, pl.ds(s * rows_per_tile, rows_per_tile)])

    return sc_edge


def kernel(feat_src, feat_dst, edge_weight, edge_index, weight_e, u, v,
           ln_gamma, ln_beta):
    n, d_in = feat_src.shape
    e, d_edge = edge_weight.shape
    d = u.shape[1]

    nsec_blocks = (n + _SEC_BLK - 1) // _SEC_BLK
    n_sec = nsec_blocks * _SEC_BLK
    e_blocks = e // _SEC_BLK
    grid = 3 * nsec_blocks + e_blocks
    t_rows = 2 * n_sec + e

    nb = nsec_blocks - 1
    t_arr, z = pl.pallas_call(
        _mega_body,
        grid=(grid,),
        in_specs=[
            pl.BlockSpec((_SEC_BLK, d_in),
                         lambda i: (jnp.clip(i - 4, 0, nb), 0)),
            pl.BlockSpec((_SEC_BLK, d_in),
                         lambda i: (jnp.clip(jnp.where(i < 4, i, i - 8),
                                             0, nb), 0)),
            pl.BlockSpec((_SEC_BLK, d_edge),
                         lambda i: (jnp.clip(i - 12, 0, e // _SEC_BLK - 1),
                                    0)),
            pl.BlockSpec((d_edge, d), lambda i: (0, 0)),
            pl.BlockSpec((d_in, d), lambda i: (0, 0)),
            pl.BlockSpec((d_in, d), lambda i: (0, 0)),
        ],
        out_specs=[
            pl.BlockSpec((_SEC_BLK, d), lambda i: (jnp.maximum(i - 4, 0), 0)),
            pl.BlockSpec((_SEC_BLK, d), lambda i: (jnp.minimum(i, nb), 0)),
        ],
        out_shape=[
            jax.ShapeDtypeStruct((t_rows, d), jnp.float32),
            jax.ShapeDtypeStruct((n_sec, d), jnp.float32),
        ],
    )(feat_src, feat_dst, edge_weight, weight_e, u, v)

    sc_edge = _make_sc_edge_call(n, e, d, n_sec)
    partials = sc_edge(t_arr, edge_index[0], edge_index[1])

    nblk = 1000
    out = pl.pallas_call(
        _final_body,
        grid=(n // nblk,),
        in_specs=[
            pl.BlockSpec((nblk, d), lambda i: (i, 0)),
            pl.BlockSpec((2, nblk, d), lambda i: (0, i, 0)),
            pl.BlockSpec((nblk, d), lambda i: (i, 0)),
            pl.BlockSpec((1, d), lambda i: (0, 0)),
            pl.BlockSpec((1, d), lambda i: (0, 0)),
        ],
        out_specs=pl.BlockSpec((nblk, d), lambda i: (i, 0)),
        out_shape=jax.ShapeDtypeStruct((n, d), jnp.float32),
    )(z, partials, feat_dst, ln_gamma.reshape(1, d), ln_beta.reshape(1, d))

    return out


# speedup vs baseline: 1.0791x; 1.0791x over previous
import functools

import jax
import jax.numpy as jnp
from jax import lax
from jax.experimental import pallas as pl
from jax.experimental.pallas import tpu as pltpu
from jax.experimental.pallas import tpu_sc as plsc

_SEC_BLK = 6400


def _mega_body(nsb, fs_ref, fd_ref, ewt_ref, we_ref, u_ref, v_ref,
               t_ref, z_ref):
    i = pl.program_id(0)

    @pl.when(i < nsb)
    def _():
        z_ref[...] = jnp.dot(fd_ref[...], u_ref[...],
                             preferred_element_type=jnp.float32)

    @pl.when((i >= nsb) & (i < 2 * nsb))
    def _():
        t_ref[...] = jnp.dot(fs_ref[...], u_ref[...],
                             preferred_element_type=jnp.float32)

    @pl.when((i >= 2 * nsb) & (i < 3 * nsb))
    def _():
        t_ref[...] = jnp.dot(fd_ref[...], v_ref[...],
                             preferred_element_type=jnp.float32)

    @pl.when(i >= 3 * nsb)
    def _():
        t_ref[...] = jnp.dot(ewt_ref[...], we_ref[...],
                             preferred_element_type=jnp.float32)


def _final_body(z_ref, p_ref, fd_ref, g_ref, b_ref, o_ref):
    x = z_ref[...] + p_ref[0] + p_ref[1]
    mean = jnp.mean(x, axis=-1, keepdims=True)
    xc = x - mean
    var = jnp.mean(xc * xc, axis=-1, keepdims=True)
    y = xc * lax.rsqrt(var + 1e-5)
    o_ref[...] = y * g_ref[...] + b_ref[...] + fd_ref[...]


def _make_sc_edge_call(n_nodes, n_edges, d, n_sec):
    info = plsc.get_sparse_core_info()
    nc, ns, lanes = info.num_cores, info.num_subcores, info.num_lanes
    nw = nc * ns
    assert n_edges % nw == 0
    epw = n_edges // nw
    blk = 16
    assert blk == lanes
    nchunk = epw // blk
    n_pad = ((n_nodes + ns * 128 - 1) // (ns * 128)) * (ns * 128)
    rows_per_tile = n_pad // ns
    zrows = 40
    assert rows_per_tile % zrows == 0
    nvec = d // lanes
    nring = 3
    hd_base = n_sec
    ew_base = 2 * n_sec

    nsteady = (nchunk - (nring + 1)) // nring
    tail = list(range(nring * nsteady, nchunk))

    mesh = plsc.VectorSubcoreMesh(core_axis_name="c", subcore_axis_name="s")

    scratch = (
        [pltpu.VMEM((epw,), jnp.int32)] * 2 +
        [pltpu.VMEM((3 * blk, d), jnp.float32)] * nring +
        [pltpu.VMEM((3 * blk,), jnp.int32)] * nring +
        [pltpu.VMEM((blk,), jnp.int32)] * nring +
        [pltpu.VMEM_SHARED((n_pad, d), jnp.float32)] +
        [pltpu.SemaphoreType.DMA] * (2 * nring)
    )

    @functools.partial(
        pl.kernel,
        out_type=jax.ShapeDtypeStruct((nc, n_pad, d), jnp.float32),
        mesh=mesh,
        scratch_types=scratch,
    )
    def sc_edge(t_hbm, src_hbm, dst_hbm, out_hbm, *scr):
        src_v, dst_v = scr[0], scr[1]
        bufs = scr[2:2 + nring]
        ibufs = scr[2 + nring:2 + 2 * nring]
        dbufs = scr[2 + 2 * nring:2 + 3 * nring]
        aggr_sh = scr[2 + 3 * nring]
        sems = scr[3 + 3 * nring:3 + 4 * nring]
        ssems = scr[3 + 4 * nring:3 + 5 * nring]

        c = lax.axis_index("c")
        s = lax.axis_index("s")
        wid = c * ns + s
        base_edge = wid * epw

        @plsc.parallel_loop(0, zrows)
        def _zero_row(i):
            for j in range(nvec):
                bufs[0][i, pl.ds(j * lanes, lanes)] = jnp.zeros(
                    (lanes,), jnp.float32)
        for k in range(rows_per_tile // zrows):
            pltpu.sync_copy(
                bufs[0].at[pl.ds(0, zrows)],
                aggr_sh.at[pl.ds(s * rows_per_tile + k * zrows, zrows)])
        plsc.subcore_barrier()

        pltpu.sync_copy(src_hbm.at[pl.ds(base_edge, epw)], src_v)
        pltpu.sync_copy(dst_hbm.at[pl.ds(base_edge, epw)], dst_v)

        iota16 = lax.iota(jnp.int32, blk)

        def drain_scatter(b):
            pltpu.make_async_copy(bufs[b].at[pl.ds(blk, blk)],
                                  aggr_sh.at[dbufs[b]], ssems[b]).wait()

        def start(ch, b):
            s_idx = src_v[pl.ds(ch * blk, blk)]
            d_idx = dst_v[pl.ds(ch * blk, blk)]
            ibufs[b][pl.ds(0, blk)] = s_idx
            ibufs[b][pl.ds(blk, blk)] = d_idx + hd_base
            ibufs[b][pl.ds(2 * blk, blk)] = (
                iota16 + (ew_base + base_edge + ch * blk))
            dbufs[b][...] = d_idx
            pltpu.async_copy(t_hbm.at[ibufs[b]], bufs[b], sems[b])

        def wait(b):
            pltpu.make_async_copy(t_hbm.at[ibufs[b]], bufs[b],
                                  sems[b]).wait()

        def process(b):
            buf = bufs[b]

            @plsc.parallel_loop(0, blk, unroll=2)
            def _row(i):
                for j in range(nvec):
                    sl = pl.ds(j * lanes, lanes)
                    a = buf[i, sl]
                    x = a * buf[blk + i, sl] * buf[2 * blk + i, sl]
                    r = 1.0 + jnp.exp(-x)
                    buf[blk + i, sl] = a / r

            pltpu.async_copy(buf.at[pl.ds(blk, blk)],
                             aggr_sh.at[dbufs[b]], ssems[b], add=True)

        for q in range(nring - 1):
            start(q, q)

        for q in range(nring):
            wait(q)
            nxt = q + nring - 1
            nslot = (q + nring - 1) % nring
            if nxt >= nring:
                drain_scatter(nslot)
            start(nxt, nslot)
            process(q)

        def ring_body(t, carry):
            for q in range(nring):
                ch = nring * t + q
                wait(q)
                nslot = (q + nring - 1) % nring
                drain_scatter(nslot)
                start(ch + nring - 1, nslot)
                process(q)
            return carry
        lax.fori_loop(1, nsteady, ring_body, 0)

        for ch in tail:
            b = ch % nring
            wait(b)
            nxt = ch + nring - 1
            if nxt < nchunk:
                nslot = nxt % nring
                drain_scatter(nslot)
                start(nxt, nslot)
            process(b)

        for b in range(nring):
            drain_scatter(b)

        plsc.subcore_barrier()
        pltpu.sync_copy(
            aggr_sh.at[pl.ds(s * rows_per_tile, rows_per_tile)],
            out_hbm.at[c, pl.ds(s * rows_per_tile, rows_per_tile)])

    return sc_edge


def kernel(feat_src, feat_dst, edge_weight, edge_index, weight_e, u, v,
           ln_gamma, ln_beta):
    n, d_in = feat_src.shape
    e, d_edge = edge_weight.shape
    d = u.shape[1]

    nsec_blocks = (n + _SEC_BLK - 1) // _SEC_BLK
    n_sec = nsec_blocks * _SEC_BLK
    e_blocks = e // _SEC_BLK
    grid = 3 * nsec_blocks + e_blocks
    t_rows = 2 * n_sec + e

    nb = nsec_blocks - 1
    nsb = nsec_blocks
    t_arr, z = pl.pallas_call(
        functools.partial(_mega_body, nsb),
        grid=(grid,),
        in_specs=[
            pl.BlockSpec((_SEC_BLK, d_in),
                         lambda i: (jnp.clip(i - nsb, 0, nb), 0)),
            pl.BlockSpec((_SEC_BLK, d_in),
                         lambda i: (jnp.clip(jnp.where(i < nsb, i,
                                                       i - 2 * nsb),
                                             0, nb), 0)),
            pl.BlockSpec((_SEC_BLK, d_edge),
                         lambda i: (jnp.clip(i - 3 * nsb, 0,
                                             e // _SEC_BLK - 1), 0)),
            pl.BlockSpec((d_edge, d), lambda i: (0, 0)),
            pl.BlockSpec((d_in, d), lambda i: (0, 0)),
            pl.BlockSpec((d_in, d), lambda i: (0, 0)),
        ],
        out_specs=[
            pl.BlockSpec((_SEC_BLK, d),
                         lambda i: (jnp.maximum(i - nsb, 0), 0)),
            pl.BlockSpec((_SEC_BLK, d), lambda i: (jnp.minimum(i, nb), 0)),
        ],
        out_shape=[
            jax.ShapeDtypeStruct((t_rows, d), jnp.float32),
            jax.ShapeDtypeStruct((n_sec, d), jnp.float32),
        ],
    )(feat_src, feat_dst, edge_weight, weight_e, u, v)

    sc_edge = _make_sc_edge_call(n, e, d, n_sec)
    partials = sc_edge(t_arr, edge_index[0], edge_index[1])

    nblk = 1000
    out = pl.pallas_call(
        _final_body,
        grid=(n // nblk,),
        in_specs=[
            pl.BlockSpec((nblk, d), lambda i: (i, 0)),
            pl.BlockSpec((2, nblk, d), lambda i: (0, i, 0)),
            pl.BlockSpec((nblk, d), lambda i: (i, 0)),
            pl.BlockSpec((1, d), lambda i: (0, 0)),
            pl.BlockSpec((1, d), lambda i: (0, 0)),
        ],
        out_specs=pl.BlockSpec((nblk, d), lambda i: (i, 0)),
        out_shape=jax.ShapeDtypeStruct((n, d), jnp.float32),
    )(z, partials, feat_dst, ln_gamma.reshape(1, d), ln_beta.reshape(1, d))

    return out


# speedup vs baseline: 1.0815x; 1.0022x over previous
import functools

import jax
import jax.numpy as jnp
import numpy as np
from jax import lax
from jax.experimental import pallas as pl
from jax.experimental.pallas import tpu as pltpu
from jax.experimental.pallas import tpu_sc as plsc

_SEC_BLK = 6400


def _mega_body(nsb, fs_ref, fd_ref, ewt_ref, we_ref, u_ref, v_ref,
               t_ref, z_ref):
    i = pl.program_id(0)

    @pl.when(i < nsb)
    def _():
        z_ref[...] = jnp.dot(fd_ref[...], u_ref[...],
                             preferred_element_type=jnp.float32)

    @pl.when((i >= nsb) & (i < 2 * nsb))
    def _():
        t_ref[...] = jnp.dot(fs_ref[...], u_ref[...],
                             preferred_element_type=jnp.float32)

    @pl.when((i >= 2 * nsb) & (i < 3 * nsb))
    def _():
        t_ref[...] = jnp.dot(fd_ref[...], v_ref[...],
                             preferred_element_type=jnp.float32)

    @pl.when(i >= 3 * nsb)
    def _():
        t_ref[...] = jnp.dot(ewt_ref[...], we_ref[...],
                             preferred_element_type=jnp.float32)


def _final_body(z_ref, p_ref, fd_ref, g_ref, b_ref, o_ref):
    x = z_ref[...] + p_ref[0] + p_ref[1]
    mean = jnp.mean(x, axis=-1, keepdims=True)
    xc = x - mean
    var = jnp.mean(xc * xc, axis=-1, keepdims=True)
    y = xc * lax.rsqrt(var + 1e-5)
    o_ref[...] = y * g_ref[...] + b_ref[...] + fd_ref[...]


def _make_sc_edge_call(n_nodes, n_edges, d, n_sec):
    info = plsc.get_sparse_core_info()
    nc, ns, lanes = info.num_cores, info.num_subcores, info.num_lanes
    nw = nc * ns
    assert n_edges % nw == 0
    epw = n_edges // nw
    blk = 16
    assert blk == lanes
    nchunk = epw // blk
    n_pad = ((n_nodes + ns * 128 - 1) // (ns * 128)) * (ns * 128)
    rows_per_tile = n_pad // ns
    zrows = 40
    assert rows_per_tile % zrows == 0
    nvec = d // lanes
    nring = 3
    hd_base = n_sec
    ew_base = 2 * n_sec

    nsteady = (nchunk - (nring + 1)) // nring
    tail = list(range(nring * nsteady, nchunk))

    mesh = plsc.VectorSubcoreMesh(core_axis_name="c", subcore_axis_name="s")

    scratch = (
        [pltpu.VMEM((epw,), jnp.int32)] * 2 +
        [pltpu.VMEM((3 * blk, d), jnp.float32)] * nring +
        [pltpu.VMEM((3 * blk,), jnp.int32)] * nring +
        [pltpu.VMEM((blk,), jnp.int32)] * nring +
        [pltpu.VMEM_SHARED((n_pad, d), jnp.float32)] +
        [pltpu.SemaphoreType.DMA] * (2 * nring)
    )

    @functools.partial(
        pl.kernel,
        out_type=jax.ShapeDtypeStruct((nc, n_pad, d), jnp.float32),
        mesh=mesh,
        scratch_types=scratch,
    )
    def sc_edge(t_hbm, src_hbm, dst_hbm, out_hbm, *scr):
        src_v, dst_v = scr[0], scr[1]
        bufs = scr[2:2 + nring]
        ibufs = scr[2 + nring:2 + 2 * nring]
        dbufs = scr[2 + 2 * nring:2 + 3 * nring]
        aggr_sh = scr[2 + 3 * nring]
        sems = scr[3 + 3 * nring:3 + 4 * nring]
        ssems = scr[3 + 4 * nring:3 + 5 * nring]

        c = lax.axis_index("c")
        s = lax.axis_index("s")
        wid = c * ns + s
        base_edge = wid * epw

        @plsc.parallel_loop(0, zrows)
        def _zero_row(i):
            for j in range(nvec):
                bufs[0][i, pl.ds(j * lanes, lanes)] = jnp.zeros(
                    (lanes,), jnp.float32)
        for k in range(rows_per_tile // zrows):
            pltpu.sync_copy(
                bufs[0].at[pl.ds(0, zrows)],
                aggr_sh.at[pl.ds(s * rows_per_tile + k * zrows, zrows)])
        plsc.subcore_barrier()

        pltpu.sync_copy(src_hbm.at[pl.ds(base_edge, epw)], src_v)
        pltpu.sync_copy(dst_hbm.at[pl.ds(base_edge, epw)], dst_v)

        iota16 = lax.iota(jnp.int32, blk)

        def drain_scatter(b):
            pltpu.make_async_copy(bufs[b].at[pl.ds(blk, blk)],
                                  aggr_sh.at[dbufs[b]], ssems[b]).wait()

        def start(ch, b):
            s_idx = src_v[pl.ds(ch * blk, blk)]
            d_idx = dst_v[pl.ds(ch * blk, blk)]
            ibufs[b][pl.ds(0, blk)] = s_idx
            ibufs[b][pl.ds(blk, blk)] = d_idx + hd_base
            ibufs[b][pl.ds(2 * blk, blk)] = (
                iota16 + (ew_base + base_edge + ch * blk))
            dbufs[b][...] = d_idx
            pltpu.async_copy(t_hbm.at[ibufs[b]], bufs[b], sems[b])

        def wait(b):
            pltpu.make_async_copy(t_hbm.at[ibufs[b]], bufs[b],
                                  sems[b]).wait()

        def process(b):
            buf = bufs[b]

            @plsc.parallel_loop(0, blk, unroll=2)
            def _row(i):
                for j in range(nvec):
                    sl = pl.ds(j * lanes, lanes)
                    a = buf[i, sl]
                    x = a * buf[blk + i, sl] * buf[2 * blk + i, sl]
                    r = 1.0 + jnp.exp(-x)
                    buf[blk + i, sl] = a / r

            pltpu.async_copy(buf.at[pl.ds(blk, blk)],
                             aggr_sh.at[dbufs[b]], ssems[b], add=True)

        for q in range(nring - 1):
            start(q, q)

        for q in range(nring):
            wait(q)
            nxt = q + nring - 1
            nslot = (q + nring - 1) % nring
            if nxt >= nring:
                drain_scatter(nslot)
            start(nxt, nslot)
            process(q)

        def ring_body(t, carry):
            for q in range(nring):
                ch = nring * t + q
                wait(q)
                nslot = (q + nring - 1) % nring
                drain_scatter(nslot)
                start(ch + nring - 1, nslot)
                process(q)
            return carry
        lax.fori_loop(1, nsteady, ring_body, 0)

        for ch in tail:
            b = ch % nring
            wait(b)
            nxt = ch + nring - 1
            if nxt < nchunk:
                nslot = nxt % nring
                drain_scatter(nslot)
                start(nxt, nslot)
            process(b)

        for b in range(nring):
            drain_scatter(b)

        plsc.subcore_barrier()
        pltpu.sync_copy(
            aggr_sh.at[pl.ds(s * rows_per_tile, rows_per_tile)],
            out_hbm.at[c, pl.ds(s * rows_per_tile, rows_per_tile)])

    return sc_edge


def kernel(feat_src, feat_dst, edge_weight, edge_index, weight_e, u, v,
           ln_gamma, ln_beta):
    n, d_in = feat_src.shape
    e, d_edge = edge_weight.shape
    d = u.shape[1]

    nsec_blocks = (n + _SEC_BLK - 1) // _SEC_BLK
    n_sec = nsec_blocks * _SEC_BLK
    e_blocks = e // _SEC_BLK
    grid = 3 * nsec_blocks + e_blocks
    t_rows = 2 * n_sec + e

    nb = nsec_blocks - 1
    nsb = nsec_blocks
    t_arr, z = pl.pallas_call(
        functools.partial(_mega_body, nsb),
        grid=(grid,),
        in_specs=[
            pl.BlockSpec((_SEC_BLK, d_in),
                         lambda i: (jnp.clip(i - nsb, 0, nb), 0)),
            pl.BlockSpec((_SEC_BLK, d_in),
                         lambda i: (jnp.clip(jnp.where(i < nsb, i,
                                                       i - 2 * nsb),
                                             0, nb), 0)),
            pl.BlockSpec((_SEC_BLK, d_edge),
                         lambda i: (jnp.clip(i - 3 * nsb, 0,
                                             e // _SEC_BLK - 1), 0)),
            pl.BlockSpec((d_edge, d), lambda i: (0, 0)),
            pl.BlockSpec((d_in, d), lambda i: (0, 0)),
            pl.BlockSpec((d_in, d), lambda i: (0, 0)),
        ],
        out_specs=[
            pl.BlockSpec((_SEC_BLK, d),
                         lambda i: (jnp.maximum(i - nsb, 0), 0)),
            pl.BlockSpec((_SEC_BLK, d), lambda i: (jnp.minimum(i, nb), 0)),
        ],
        out_shape=[
            jax.ShapeDtypeStruct((t_rows, d), jnp.float32),
            jax.ShapeDtypeStruct((n_sec, d), jnp.float32),
        ],
    )(feat_src, feat_dst, edge_weight, weight_e, u, v)

    sc_edge = _make_sc_edge_call(n, e, d, n_sec)
    partials = sc_edge(t_arr, edge_index[0], edge_index[1])

    nblk = 1000
    out = pl.pallas_call(
        _final_body,
        grid=(n // nblk,),
        in_specs=[
            pl.BlockSpec((nblk, d), lambda i: (i, 0)),
            pl.BlockSpec((2, nblk, d), lambda i: (0, i, 0)),
            pl.BlockSpec((nblk, d), lambda i: (i, 0)),
            pl.BlockSpec((1, d), lambda i: (0, 0)),
            pl.BlockSpec((1, d), lambda i: (0, 0)),
        ],
        out_specs=pl.BlockSpec((nblk, d), lambda i: (i, 0)),
        out_shape=jax.ShapeDtypeStruct((n, d), jnp.float32),
    )(z, partials, feat_dst, ln_gamma.reshape(1, d), ln_beta.reshape(1, d))

    return out


# speedup vs baseline: 1.1984x; 1.1082x over previous
import functools

import jax
import jax.numpy as jnp
import numpy as np
from jax import lax
from jax.experimental import pallas as pl
from jax.experimental.pallas import tpu as pltpu
from jax.experimental.pallas import tpu_sc as plsc

_SEC_BLK = 6400


def _mega_body(nsb, fs_ref, fd_ref, ewt_ref, we_ref, u_ref, v_ref,
               t_ref, z_ref):
    i = pl.program_id(0)

    @pl.when(i < nsb)
    def _():
        z_ref[...] = jnp.dot(fd_ref[...], u_ref[...],
                             preferred_element_type=jnp.float32)

    @pl.when((i >= nsb) & (i < 2 * nsb))
    def _():
        t_ref[...] = jnp.dot(fs_ref[...], u_ref[...],
                             preferred_element_type=jnp.float32)

    @pl.when((i >= 2 * nsb) & (i < 3 * nsb))
    def _():
        t_ref[...] = jnp.dot(fd_ref[...], v_ref[...],
                             preferred_element_type=jnp.float32)

    @pl.when(i >= 3 * nsb)
    def _():
        t_ref[...] = jnp.dot(ewt_ref[...], we_ref[...],
                             preferred_element_type=jnp.float32)


def _final_body(z_ref, p_ref, fd_ref, g_ref, b_ref, o_ref):
    x = z_ref[...] + p_ref[0] + p_ref[1]
    mean = jnp.mean(x, axis=-1, keepdims=True)
    xc = x - mean
    var = jnp.mean(xc * xc, axis=-1, keepdims=True)
    y = xc * lax.rsqrt(var + 1e-5)
    o_ref[...] = y * g_ref[...] + b_ref[...] + fd_ref[...]


def _make_sc_edge_call(n_nodes, n_edges, d, n_sec):
    info = plsc.get_sparse_core_info()
    nc, ns, lanes = info.num_cores, info.num_subcores, info.num_lanes
    nw = nc * ns
    assert n_edges % nw == 0
    epw = n_edges // nw
    blk = 16
    assert blk == lanes
    nchunk = epw // blk
    n_pad = ((n_nodes + ns * 128 - 1) // (ns * 128)) * (ns * 128)
    rows_per_tile = n_pad // ns
    zrows = 40
    assert rows_per_tile % zrows == 0
    nvec = d // lanes
    nring = 4
    hd_base = n_sec
    ew_base = 2 * n_sec

    nsteady = (nchunk - (nring + 1)) // nring
    tail = list(range(nring * nsteady, nchunk))

    mesh = plsc.VectorSubcoreMesh(core_axis_name="c", subcore_axis_name="s")

    scratch = (
        [pltpu.VMEM((epw,), jnp.int32)] * 2 +
        [pltpu.VMEM((3 * blk, d), jnp.float32)] * nring +
        [pltpu.VMEM((3 * blk,), jnp.int32)] * nring +
        [pltpu.VMEM((blk,), jnp.int32)] * nring +
        [pltpu.VMEM_SHARED((n_pad, d), jnp.float32)] +
        [pltpu.SemaphoreType.DMA] * (2 * nring)
    )

    @functools.partial(
        pl.kernel,
        out_type=jax.ShapeDtypeStruct((nc, n_pad, d), jnp.float32),
        mesh=mesh,
        scratch_types=scratch,
    )
    def sc_edge(t_hbm, src_hbm, dst_hbm, out_hbm, *scr):
        src_v, dst_v = scr[0], scr[1]
        bufs = scr[2:2 + nring]
        ibufs = scr[2 + nring:2 + 2 * nring]
        dbufs = scr[2 + 2 * nring:2 + 3 * nring]
        aggr_sh = scr[2 + 3 * nring]
        sems = scr[3 + 3 * nring:3 + 4 * nring]
        ssems = scr[3 + 4 * nring:3 + 5 * nring]

        c = lax.axis_index("c")
        s = lax.axis_index("s")
        wid = c * ns + s
        base_edge = wid * epw

        @plsc.parallel_loop(0, zrows)
        def _zero_row(i):
            for j in range(nvec):
                bufs[0][i, pl.ds(j * lanes, lanes)] = jnp.zeros(
                    (lanes,), jnp.float32)
        for k in range(rows_per_tile // zrows):
            pltpu.sync_copy(
                bufs[0].at[pl.ds(0, zrows)],
                aggr_sh.at[pl.ds(s * rows_per_tile + k * zrows, zrows)])
        plsc.subcore_barrier()

        pltpu.sync_copy(src_hbm.at[pl.ds(base_edge, epw)], src_v)
        pltpu.sync_copy(dst_hbm.at[pl.ds(base_edge, epw)], dst_v)

        iota16 = lax.iota(jnp.int32, blk)

        def drain_scatter(b):
            pltpu.make_async_copy(bufs[b].at[pl.ds(blk, blk)],
                                  aggr_sh.at[dbufs[b]], ssems[b]).wait()

        def start(ch, b):
            s_idx = src_v[pl.ds(ch * blk, blk)]
            d_idx = dst_v[pl.ds(ch * blk, blk)]
            ibufs[b][pl.ds(0, blk)] = s_idx
            ibufs[b][pl.ds(blk, blk)] = d_idx + hd_base
            ibufs[b][pl.ds(2 * blk, blk)] = (
                iota16 + (ew_base + base_edge + ch * blk))
            dbufs[b][...] = d_idx
            pltpu.async_copy(t_hbm.at[ibufs[b]], bufs[b], sems[b])

        def wait(b):
            pltpu.make_async_copy(t_hbm.at[ibufs[b]], bufs[b],
                                  sems[b]).wait()

        def process(b):
            buf = bufs[b]

            @plsc.parallel_loop(0, blk, unroll=2)
            def _row(i):
                for j in range(nvec):
                    sl = pl.ds(j * lanes, lanes)
                    a = buf[i, sl]
                    x = a * buf[blk + i, sl] * buf[2 * blk + i, sl]
                    r = 1.0 + jnp.exp(-x)
                    buf[blk + i, sl] = a / r

            pltpu.async_copy(buf.at[pl.ds(blk, blk)],
                             aggr_sh.at[dbufs[b]], ssems[b], add=True)

        for q in range(nring - 1):
            start(q, q)

        for q in range(nring):
            wait(q)
            nxt = q + nring - 1
            nslot = (q + nring - 1) % nring
            if nxt >= nring:
                drain_scatter(nslot)
            start(nxt, nslot)
            process(q)

        def ring_body(t, carry):
            for q in range(nring):
                ch = nring * t + q
                wait(q)
                nslot = (q + nring - 1) % nring
                drain_scatter(nslot)
                start(ch + nring - 1, nslot)
                process(q)
            return carry
        lax.fori_loop(1, nsteady, ring_body, 0)

        for ch in tail:
            b = ch % nring
            wait(b)
            nxt = ch + nring - 1
            if nxt < nchunk:
                nslot = nxt % nring
                drain_scatter(nslot)
                start(nxt, nslot)
            process(b)

        for b in range(nring):
            drain_scatter(b)

        plsc.subcore_barrier()
        pltpu.sync_copy(
            aggr_sh.at[pl.ds(s * rows_per_tile, rows_per_tile)],
            out_hbm.at[c, pl.ds(s * rows_per_tile, rows_per_tile)])

    return sc_edge


def kernel(feat_src, feat_dst, edge_weight, edge_index, weight_e, u, v,
           ln_gamma, ln_beta):
    n, d_in = feat_src.shape
    e, d_edge = edge_weight.shape
    d = u.shape[1]

    nsec_blocks = (n + _SEC_BLK - 1) // _SEC_BLK
    n_sec = nsec_blocks * _SEC_BLK
    e_blocks = e // _SEC_BLK
    grid = 3 * nsec_blocks + e_blocks
    t_rows = 2 * n_sec + e

    nb = nsec_blocks - 1
    nsb = nsec_blocks
    t_arr, z = pl.pallas_call(
        functools.partial(_mega_body, nsb),
        grid=(grid,),
        in_specs=[
            pl.BlockSpec((_SEC_BLK, d_in),
                         lambda i: (jnp.clip(i - nsb, 0, nb), 0)),
            pl.BlockSpec((_SEC_BLK, d_in),
                         lambda i: (jnp.clip(jnp.where(i < nsb, i,
                                                       i - 2 * nsb),
                                             0, nb), 0)),
            pl.BlockSpec((_SEC_BLK, d_edge),
                         lambda i: (jnp.clip(i - 3 * nsb, 0,
                                             e // _SEC_BLK - 1), 0)),
            pl.BlockSpec((d_edge, d), lambda i: (0, 0)),
            pl.BlockSpec((d_in, d), lambda i: (0, 0)),
            pl.BlockSpec((d_in, d), lambda i: (0, 0)),
        ],
        out_specs=[
            pl.BlockSpec((_SEC_BLK, d),
                         lambda i: (jnp.maximum(i - nsb, 0), 0)),
            pl.BlockSpec((_SEC_BLK, d), lambda i: (jnp.minimum(i, nb), 0)),
        ],
        out_shape=[
            jax.ShapeDtypeStruct((t_rows, d), jnp.float32),
            jax.ShapeDtypeStruct((n_sec, d), jnp.float32),
        ],
    )(feat_src, feat_dst, edge_weight, weight_e, u, v)

    sc_edge = _make_sc_edge_call(n, e, d, n_sec)
    partials = sc_edge(t_arr, edge_index[0], edge_index[1])

    nblk = 1000
    out = pl.pallas_call(
        _final_body,
        grid=(n // nblk,),
        in_specs=[
            pl.BlockSpec((nblk, d), lambda i: (i, 0)),
            pl.BlockSpec((2, nblk, d), lambda i: (0, i, 0)),
            pl.BlockSpec((nblk, d), lambda i: (i, 0)),
            pl.BlockSpec((1, d), lambda i: (0, 0)),
            pl.BlockSpec((1, d), lambda i: (0, 0)),
        ],
        out_specs=pl.BlockSpec((nblk, d), lambda i: (i, 0)),
        out_shape=jax.ShapeDtypeStruct((n, d), jnp.float32),
    )(z, partials, feat_dst, ln_gamma.reshape(1, d), ln_beta.reshape(1, d))

    return out
